# baseline (speedup 1.0000x reference)
import functools

import jax
import jax.numpy as jnp
from jax import lax
from jax.experimental import pallas as pl
from jax.experimental.pallas import tpu as pltpu
from jax.experimental.pallas import tpu_sc as plsc

_N_WORDS = 100000
_DIM = 16
_CPR = 16

_mesh = plsc.VectorSubcoreMesh(core_axis_name="c", subcore_axis_name="s")


@functools.partial(
    pl.kernel,
    out_type=(),
    mesh=_mesh,
    compiler_params=pltpu.CompilerParams(use_tc_tiling_on_sc=False),
    scratch_types=[
        pltpu.VMEM((1,), jnp.int32),
        pltpu.VMEM((1,), jnp.int32),
        pltpu.VMEM((_DIM,), jnp.int32),
        pltpu.VMEM((_DIM,), jnp.float32),
        pltpu.VMEM((1, _DIM), jnp.float32),
        pltpu.VMEM((1, _CPR), jnp.int32),
        pltpu.VMEM((1, _DIM, _DIM), jnp.float32),
        pltpu.SemaphoreType.DMA,
    ],
)
def _sc_update(idx_hbm, idxc_hbm, rem_hbm, vec_hbm, sum_ref, cnt_ref, cov_ref,
               idx_v, idxc_v, rem_v, vec_v, sum_v, cnt_v, cov_v, sem):
    cid = lax.axis_index("c")
    sid = lax.axis_index("s")

    @pl.when(jnp.logical_and(cid == 0, sid == 0))
    def _():
        pltpu.sync_copy(idx_hbm, idx_v)
        pltpu.sync_copy(idxc_hbm, idxc_v)
        pltpu.sync_copy(rem_hbm, rem_v)
        pltpu.sync_copy(vec_hbm, vec_v)
        pltpu.async_copy(sum_ref.at[idx_v], sum_v, sem).wait()
        pltpu.async_copy(cnt_ref.at[idxc_v], cnt_v, sem).wait()
        pltpu.async_copy(cov_ref.at[idx_v], cov_v, sem).wait()
        vec = vec_v[...]
        sum_v[0, :] = sum_v[0, :] + vec
        lane = lax.iota(jnp.int32, _CPR)
        cnt_v[0, :] = cnt_v[0, :] + jnp.where(lane == rem_v[...], 1, 0)
        for j in range(_DIM):
            cov_v[0, j, :] = cov_v[0, j, :] + vec * vec[j]
        pltpu.async_copy(sum_v, sum_ref.at[idx_v], sem).wait()
        pltpu.async_copy(cnt_v, cnt_ref.at[idxc_v], sem).wait()
        pltpu.async_copy(cov_v, cov_ref.at[idx_v], sem).wait()


def kernel(ix, vec, sum_buf, counts, cov_buf):
    ix32 = jnp.asarray(ix, jnp.int32)
    idx = jnp.reshape(ix32, (1,))
    idxc = jnp.reshape(ix32 // _CPR, (1,))
    rem = jnp.full((_CPR,), ix32 % _CPR, jnp.int32)
    sum_ref = jax.new_ref(sum_buf)
    cnt_ref = jax.new_ref(counts.reshape(_N_WORDS // _CPR, _CPR))
    cov_ref = jax.new_ref(cov_buf)
    _sc_update(idx, idxc, rem, vec, sum_ref, cnt_ref, cov_ref)
    return sum_ref[...], cnt_ref[...].reshape(_N_WORDS), cov_ref[...]


# speedup vs baseline: 6.1994x; 6.1994x over previous
import functools

import jax
import jax.numpy as jnp
from jax import lax
from jax.experimental import pallas as pl
from jax.experimental.pallas import tpu as pltpu
from jax.experimental.pallas import tpu_sc as plsc

_N_WORDS = 100000
_DIM = 16
_CPAD = 96

_mesh = plsc.VectorSubcoreMesh(core_axis_name="c", subcore_axis_name="s")


@functools.partial(
    pl.kernel,
    out_type=(),
    mesh=_mesh,
    scratch_types=[
        pltpu.VMEM((1,), jnp.int32),
        pltpu.VMEM((1,), jnp.int32),
        pltpu.VMEM((1,), jnp.int32),
        pltpu.VMEM((3, _DIM), jnp.int32),
        pltpu.VMEM((_DIM,), jnp.float32),
        pltpu.VMEM((1, 128), jnp.float32),
        pltpu.VMEM((1, 128), jnp.int32),
        pltpu.VMEM((1, 16 * _DIM), jnp.float32),
        pltpu.SemaphoreType.DMA,
    ],
)
def _sc_update(idxs_hbm, idxc_hbm, idxv_hbm, bc_hbm, vec_hbm,
               sum_ref, cnt_ref, cov_ref,
               idxs_v, idxc_v, idxv_v, bc_v, vec_v, sum_v, cnt_v, cov_v, sem):
    cid = lax.axis_index("c")
    sid = lax.axis_index("s")

    @pl.when(jnp.logical_and(cid == 0, sid == 0))
    def _():
        pltpu.sync_copy(idxs_hbm, idxs_v)
        pltpu.sync_copy(idxc_hbm, idxc_v)
        pltpu.sync_copy(idxv_hbm, idxv_v)
        pltpu.sync_copy(bc_hbm, bc_v)
        pltpu.sync_copy(vec_hbm, vec_v)
        pltpu.async_copy(sum_ref.at[idxs_v], sum_v, sem).wait()
        pltpu.async_copy(cnt_ref.at[idxc_v], cnt_v, sem).wait()
        pltpu.async_copy(cov_ref.at[idxv_v], cov_v, sem).wait()
        vec = vec_v[...]
        rem8 = bc_v[0, :]
        grp = bc_v[1, :]
        lane = bc_v[2, :]
        iota = lax.iota(jnp.int32, _DIM)
        zf = jnp.zeros((_DIM,), jnp.float32)
        for j in range(8):
            s = pl.ds(j * _DIM, _DIM)
            sum_v[0, s] = sum_v[0, s] + jnp.where(rem8 == j, vec, zf)
            hit = jnp.logical_and(grp == j, iota == lane)
            cnt_v[0, s] = cnt_v[0, s] + jnp.where(hit, 1, 0)
        for j in range(_DIM):
            s = pl.ds(j * _DIM, _DIM)
            cov_v[0, s] = cov_v[0, s] + vec * vec[j]
        pltpu.async_copy(sum_v, sum_ref.at[idxs_v], sem).wait()
        pltpu.async_copy(cnt_v, cnt_ref.at[idxc_v], sem).wait()
        pltpu.async_copy(cov_v, cov_ref.at[idxv_v], sem).wait()


def kernel(ix, vec, sum_buf, counts, cov_buf):
    ix32 = jnp.asarray(ix, jnp.int32)
    idxs = jnp.reshape(ix32 // 8, (1,))
    idxc = jnp.reshape(ix32 // 128, (1,))
    idxv = jnp.reshape(ix32, (1,))
    bc = jnp.stack([
        jnp.full((_DIM,), ix32 % 8, jnp.int32),
        jnp.full((_DIM,), (ix32 % 128) // _DIM, jnp.int32),
        jnp.full((_DIM,), ix32 % _DIM, jnp.int32),
    ])
    cpad = jnp.concatenate([counts, jnp.zeros((_CPAD,), jnp.int32)])
    sum_ref = jax.new_ref(sum_buf.reshape(_N_WORDS // 8, 8 * _DIM))
    cnt_ref = jax.new_ref(cpad.reshape((_N_WORDS + _CPAD) // 128, 128))
    cov_ref = jax.new_ref(cov_buf.reshape(_N_WORDS, _DIM * _DIM))
    _sc_update(idxs, idxc, idxv, bc, vec, sum_ref, cnt_ref, cov_ref)
    new_sum = sum_ref[...].reshape(_N_WORDS, _DIM)
    new_counts = cnt_ref[...].reshape(-1)[:_N_WORDS]
    new_cov = cov_ref[...].reshape(_N_WORDS, _DIM, _DIM)
    return new_sum, new_counts, new_cov
